# baseline (speedup 1.0000x reference)
import functools

import jax
import jax.numpy as jnp
from jax import lax
from jax.experimental import pallas as pl
from jax.experimental.pallas import tpu as pltpu
from jax.experimental.pallas import tpu_sc as plsc

N = 10000
NP = 10240
DF = 128
DC = 40
DCP = 64
E = 320000
NC = 2
NS = 16
NW = NC * NS
CHUNK = 128
CPW = 80
EP = NW * CPW * CHUNK
RPS = NP // NS
RB = 1024
GRID = NP // RB
NBUF = 8
CPW0 = 80
CPW1 = 80
EPA = NS * CPW0 * CHUNK
EPB = NS * CPW1 * CHUNK

_mesh = plsc.VectorSubcoreMesh(core_axis_name="c", subcore_axis_name="s")


@functools.partial(
    pl.kernel,
    out_type=jax.ShapeDtypeStruct((NC, NP), jnp.float32),
    mesh=_mesh,
    scratch_types=[
        pltpu.VMEM((CPW, CHUNK), jnp.int32),
        pltpu.VMEM((CHUNK,), jnp.float32),
        pltpu.VMEM((RPS,), jnp.float32),
        pltpu.VMEM_SHARED((NP,), jnp.float32),
        pltpu.SemaphoreType.DMA,
    ],
)
def _deg_kernel(dst_hbm, out_hbm, didx, ones_v, zb, acc, sem):
    cid = lax.axis_index("c")
    sid = lax.axis_index("s")
    wid = sid * NC + cid

    @pl.loop(0, CHUNK, step=16)
    def _(i):
        ones_v[pl.ds(i, 16)] = jnp.ones((16,), jnp.float32)

    @pl.loop(0, RPS, step=16)
    def _(i):
        zb[pl.ds(i, 16)] = jnp.zeros((16,), jnp.float32)

    pltpu.sync_copy(zb, acc.at[pl.ds(sid * RPS, RPS)])
    pltpu.async_copy(dst_hbm.at[wid], didx, sem).wait()
    plsc.subcore_barrier()

    @pl.loop(0, CPW)
    def _(j):
        pltpu.sync_copy(ones_v, acc.at[didx.at[j]], add=True)

    plsc.subcore_barrier()
    pltpu.sync_copy(acc.at[pl.ds(sid * RPS, RPS)],
                    out_hbm.at[cid, pl.ds(sid * RPS, RPS)])


def _make_scatter(D):
    CPM = max(CPW0, CPW1)

    @functools.partial(
        pl.kernel,
        out_type=jax.ShapeDtypeStruct((NC, NP, D), jnp.bfloat16),
        mesh=_mesh,
        compiler_params=pltpu.CompilerParams(use_tc_tiling_on_sc=False),
        scratch_types=[
            pltpu.VMEM((CPM, CHUNK), jnp.int32),
            pltpu.VMEM((CPM, CHUNK), jnp.int32),
            [pltpu.VMEM((CHUNK, D), jnp.bfloat16) for _ in range(NBUF)],
            pltpu.VMEM((64, D), jnp.bfloat16),
            pltpu.VMEM_SHARED((NP, D), jnp.bfloat16),
            [pltpu.SemaphoreType.DMA for _ in range(NBUF)],
            pltpu.SemaphoreType.DMA,
        ],
    )
    def _scatter(g_hbm, srca_hbm, dsta_hbm, srcb_hbm, dstb_hbm, out_hbm,
                 sidx, didx, bufs, zb, acc, sems, semi):
        cid = lax.axis_index("c")
        sid = lax.axis_index("s")

        @pl.loop(0, 64)
        def _(r):
            @pl.loop(0, D, step=32)
            def _(c):
                zb[r, pl.ds(c, 32)] = jnp.zeros((32,), jnp.bfloat16)

        @pl.loop(0, RPS // 64)
        def _(t):
            pltpu.sync_copy(zb, acc.at[pl.ds(sid * RPS + t * 64, 64)])

        def phase(src_hbm, dst_hbm, cpw):
            pltpu.async_copy(src_hbm.at[sid], sidx.at[pl.ds(0, cpw)], semi)
            pltpu.async_copy(dst_hbm.at[sid], didx.at[pl.ds(0, cpw)], semi)
            pltpu.make_async_copy(src_hbm.at[sid],
                                  sidx.at[pl.ds(0, cpw)], semi).wait()
            pltpu.make_async_copy(dst_hbm.at[sid],
                                  didx.at[pl.ds(0, cpw)], semi).wait()
            plsc.subcore_barrier()

            for k in range(NBUF):
                pltpu.async_copy(g_hbm.at[sidx.at[k]], bufs[k], sems[k])

            @pl.loop(0, cpw - NBUF, step=NBUF)
            def _(j):
                for k in range(NBUF):
                    pltpu.make_async_copy(g_hbm.at[sidx.at[j + k]],
                                          bufs[k], sems[k]).wait()
                    pltpu.sync_copy(bufs[k], acc.at[didx.at[j + k]], add=True)
                    pltpu.async_copy(g_hbm.at[sidx.at[j + NBUF + k]],
                                     bufs[k], sems[k])

            j0 = cpw - NBUF
            for k in range(NBUF):
                pltpu.make_async_copy(g_hbm.at[sidx.at[j0 + k]],
                                      bufs[k], sems[k]).wait()
                pltpu.sync_copy(bufs[k], acc.at[didx.at[j0 + k]], add=True)

        @pl.when(cid == 0)
        def _():
            phase(srca_hbm, dsta_hbm, CPW0)

        @pl.when(cid == 1)
        def _():
            phase(srcb_hbm, dstb_hbm, CPW1)

        plsc.subcore_barrier()
        pltpu.sync_copy(acc.at[pl.ds(sid * RPS, RPS)],
                        out_hbm.at[cid, pl.ds(sid * RPS, RPS)])

    return _scatter


_scatter_f = _make_scatter(DF)
_scatter_c = _make_scatter(DCP)


def _layer1_body(x_ref, w_ref, b_ref, da_ref, db_ref, g_ref, dis_ref):
    deg = da_ref[...] + db_ref[...] + 1.0
    dis = lax.rsqrt(deg)
    h = jnp.dot(x_ref[...], w_ref[...],
                preferred_element_type=jnp.float32) + b_ref[...]
    g_ref[...] = (h * dis).astype(jnp.bfloat16)
    dis_ref[...] = dis


_layer1_call = pl.pallas_call(
    _layer1_body,
    grid=(GRID,),
    in_specs=[
        pl.BlockSpec((RB, DF), lambda i: (i, 0)),
        pl.BlockSpec((DF, DF), lambda i: (0, 0)),
        pl.BlockSpec((1, DF), lambda i: (0, 0)),
        pl.BlockSpec((RB, 1), lambda i: (i, 0)),
        pl.BlockSpec((RB, 1), lambda i: (i, 0)),
    ],
    out_specs=[
        pl.BlockSpec((RB, DF), lambda i: (i, 0)),
        pl.BlockSpec((RB, 1), lambda i: (i, 0)),
    ],
    out_shape=[
        jax.ShapeDtypeStruct((NP, DF), jnp.bfloat16),
        jax.ShapeDtypeStruct((NP, 1), jnp.float32),
    ],
)


def _layer2_body(s_ref, g_ref, dis_ref, w_ref, b_ref, o_ref):
    dis = dis_ref[...]
    s1 = s_ref[0].astype(jnp.float32) + s_ref[1].astype(jnp.float32)
    agg = (s1 + g_ref[...].astype(jnp.float32)) * dis
    h = jnp.maximum(agg, 0.0)
    g2 = (jnp.dot(h, w_ref[...], preferred_element_type=jnp.float32)
          + b_ref[...]) * dis
    o_ref[...] = jnp.concatenate(
        [g2, jnp.zeros((RB, DCP - DC), jnp.float32)], axis=1
    ).astype(jnp.bfloat16)


_layer2_call = pl.pallas_call(
    _layer2_body,
    grid=(GRID,),
    in_specs=[
        pl.BlockSpec((NC, RB, DF), lambda i: (0, i, 0)),
        pl.BlockSpec((RB, DF), lambda i: (i, 0)),
        pl.BlockSpec((RB, 1), lambda i: (i, 0)),
        pl.BlockSpec((DF, DC), lambda i: (0, 0)),
        pl.BlockSpec((1, DC), lambda i: (0, 0)),
    ],
    out_specs=pl.BlockSpec((RB, DCP), lambda i: (i, 0)),
    out_shape=jax.ShapeDtypeStruct((NP, DCP), jnp.bfloat16),
)


def _out_body(s_ref, g_ref, dis_ref, o_ref):
    s2 = s_ref[0].astype(jnp.float32) + s_ref[1].astype(jnp.float32)
    z = (s2 + g_ref[...].astype(jnp.float32)) * dis_ref[...]
    col = lax.broadcasted_iota(jnp.int32, (RB, DCP), 1)
    mask = col < DC
    zm = jnp.where(mask, z, -jnp.inf)
    m = jnp.max(zm, axis=1, keepdims=True)
    e = jnp.where(mask, jnp.exp(zm - m), 0.0)
    lse = jnp.log(jnp.sum(e, axis=1, keepdims=True)) + m
    o_ref[...] = (z - lse)[:, :DC]


_out_call = pl.pallas_call(
    _out_body,
    grid=(GRID,),
    in_specs=[
        pl.BlockSpec((NC, RB, DCP), lambda i: (0, i, 0)),
        pl.BlockSpec((RB, DCP), lambda i: (i, 0)),
        pl.BlockSpec((RB, 1), lambda i: (i, 0)),
    ],
    out_specs=pl.BlockSpec((RB, DC), lambda i: (i, 0)),
    out_shape=jax.ShapeDtypeStruct((N, DC), jnp.float32),
)


def kernel(x, edge_index, W1, b1, W2, b2):
    src = edge_index[0].astype(jnp.int32)
    dst = edge_index[1].astype(jnp.int32)
    pad = EP - E
    dst_w = jnp.concatenate(
        [dst, jnp.full((pad,), N, jnp.int32)]).reshape(NW, CPW, CHUNK)
    srca = src[:EPA].reshape(NS, CPW0, CHUNK)
    dsta = dst[:EPA].reshape(NS, CPW0, CHUNK)
    padb = EPB - (E - EPA)
    srcb = jnp.concatenate(
        [src[EPA:], jnp.zeros((padb,), jnp.int32)]).reshape(NS, CPW1, CHUNK)
    dstb = jnp.concatenate(
        [dst[EPA:], jnp.full((padb,), N, jnp.int32)]).reshape(NS, CPW1, CHUNK)

    b1r = b1.reshape(1, DF)
    b2r = b2.reshape(1, DC)

    deg2 = _deg_kernel(dst_w)
    dega = deg2[0].reshape(NP, 1)
    degb = deg2[1].reshape(NP, 1)

    g1, dis = _layer1_call(x, W1, b1r, dega, degb)
    s1 = _scatter_f(g1, srca, dsta, srcb, dstb)
    g2 = _layer2_call(s1, g1, dis, W2, b2r)
    s2 = _scatter_c(g2, srca, dsta, srcb, dstb)
    return _out_call(s2, g2, dis)


# speedup vs baseline: 1.0395x; 1.0395x over previous
import functools

import jax
import jax.numpy as jnp
from jax import lax
from jax.experimental import pallas as pl
from jax.experimental.pallas import tpu as pltpu
from jax.experimental.pallas import tpu_sc as plsc

N = 10000
NP = 10240
DF = 128
DC = 40
DCP = 64
E = 320000
NC = 2
NS = 16
NW = NC * NS
CHUNK = 128
CPW = 80
EP = NW * CPW * CHUNK
RPS = NP // NS
RB = 1024
GRID = NP // RB
NBUF = 4

_mesh = plsc.VectorSubcoreMesh(core_axis_name="c", subcore_axis_name="s")


@functools.partial(
    pl.kernel,
    out_type=jax.ShapeDtypeStruct((NC, NP), jnp.float32),
    mesh=_mesh,
    scratch_types=[
        pltpu.VMEM((CPW, CHUNK), jnp.int32),
        pltpu.VMEM((CHUNK,), jnp.float32),
        pltpu.VMEM((RPS,), jnp.float32),
        pltpu.VMEM_SHARED((NP,), jnp.float32),
        pltpu.SemaphoreType.DMA,
    ],
)
def _deg_kernel(dst_hbm, out_hbm, didx, ones_v, zb, acc, sem):
    cid = lax.axis_index("c")
    sid = lax.axis_index("s")
    wid = sid * NC + cid

    @pl.loop(0, CHUNK, step=16)
    def _(i):
        ones_v[pl.ds(i, 16)] = jnp.ones((16,), jnp.float32)

    @pl.loop(0, RPS, step=16)
    def _(i):
        zb[pl.ds(i, 16)] = jnp.zeros((16,), jnp.float32)

    pltpu.sync_copy(zb, acc.at[pl.ds(sid * RPS, RPS)])
    pltpu.async_copy(dst_hbm.at[wid], didx, sem).wait()
    plsc.subcore_barrier()

    @pl.loop(0, CPW)
    def _(j):
        pltpu.sync_copy(ones_v, acc.at[didx.at[j]], add=True)

    plsc.subcore_barrier()
    pltpu.sync_copy(acc.at[pl.ds(sid * RPS, RPS)],
                    out_hbm.at[cid, pl.ds(sid * RPS, RPS)])


def _make_scatter(D):
    @functools.partial(
        pl.kernel,
        out_type=jax.ShapeDtypeStruct((NC, NP, D), jnp.bfloat16),
        mesh=_mesh,
        compiler_params=pltpu.CompilerParams(use_tc_tiling_on_sc=False),
        scratch_types=[
            pltpu.VMEM((CPW, CHUNK), jnp.int32),
            pltpu.VMEM((CPW, CHUNK), jnp.int32),
            [pltpu.VMEM((CHUNK, D), jnp.bfloat16) for _ in range(NBUF)],
            pltpu.VMEM((64, D), jnp.bfloat16),
            pltpu.VMEM_SHARED((NP, D), jnp.bfloat16),
            [pltpu.SemaphoreType.DMA for _ in range(NBUF)],
            pltpu.SemaphoreType.DMA,
        ],
    )
    def _scatter(g_hbm, src_hbm, dst_hbm, out_hbm,
                 sidx, didx, bufs, zb, acc, sems, semi):
        cid = lax.axis_index("c")
        sid = lax.axis_index("s")
        wid = sid * NC + cid

        @pl.loop(0, 64)
        def _(r):
            @pl.loop(0, D, step=32)
            def _(c):
                zb[r, pl.ds(c, 32)] = jnp.zeros((32,), jnp.bfloat16)

        @pl.loop(0, RPS // 64)
        def _(t):
            pltpu.sync_copy(zb, acc.at[pl.ds(sid * RPS + t * 64, 64)])

        pltpu.async_copy(src_hbm.at[wid], sidx, semi)
        pltpu.async_copy(dst_hbm.at[wid], didx, semi)
        pltpu.make_async_copy(src_hbm.at[wid], sidx, semi).wait()
        pltpu.make_async_copy(dst_hbm.at[wid], didx, semi).wait()
        plsc.subcore_barrier()

        for k in range(NBUF):
            pltpu.async_copy(g_hbm.at[sidx.at[k]], bufs[k], sems[k])

        @pl.loop(0, CPW - NBUF, step=NBUF)
        def _(j):
            for k in range(NBUF):
                pltpu.make_async_copy(g_hbm.at[sidx.at[j + k]],
                                      bufs[k], sems[k]).wait()
                pltpu.sync_copy(bufs[k], acc.at[didx.at[j + k]], add=True)
                pltpu.async_copy(g_hbm.at[sidx.at[j + NBUF + k]],
                                 bufs[k], sems[k])

        j0 = CPW - NBUF
        for k in range(NBUF):
            pltpu.make_async_copy(g_hbm.at[sidx.at[j0 + k]],
                                  bufs[k], sems[k]).wait()
            pltpu.sync_copy(bufs[k], acc.at[didx.at[j0 + k]], add=True)

        plsc.subcore_barrier()
        pltpu.sync_copy(acc.at[pl.ds(sid * RPS, RPS)],
                        out_hbm.at[cid, pl.ds(sid * RPS, RPS)])

    return _scatter


_scatter_f = _make_scatter(DF)
_scatter_c = _make_scatter(DCP)


def _layer1_body(x_ref, w_ref, b_ref, da_ref, db_ref, g_ref, dis_ref):
    deg = da_ref[...] + db_ref[...] + 1.0
    dis = lax.rsqrt(deg)
    h = jnp.dot(x_ref[...], w_ref[...],
                preferred_element_type=jnp.float32) + b_ref[...]
    g_ref[...] = (h * dis).astype(jnp.bfloat16)
    dis_ref[...] = dis


_layer1_call = pl.pallas_call(
    _layer1_body,
    grid=(GRID,),
    in_specs=[
        pl.BlockSpec((RB, DF), lambda i: (i, 0)),
        pl.BlockSpec((DF, DF), lambda i: (0, 0)),
        pl.BlockSpec((1, DF), lambda i: (0, 0)),
        pl.BlockSpec((RB, 1), lambda i: (i, 0)),
        pl.BlockSpec((RB, 1), lambda i: (i, 0)),
    ],
    out_specs=[
        pl.BlockSpec((RB, DF), lambda i: (i, 0)),
        pl.BlockSpec((RB, 1), lambda i: (i, 0)),
    ],
    out_shape=[
        jax.ShapeDtypeStruct((NP, DF), jnp.bfloat16),
        jax.ShapeDtypeStruct((NP, 1), jnp.float32),
    ],
)


def _layer2_body(s_ref, g_ref, dis_ref, w_ref, b_ref, o_ref):
    dis = dis_ref[...]
    s1 = s_ref[0].astype(jnp.float32) + s_ref[1].astype(jnp.float32)
    agg = (s1 + g_ref[...].astype(jnp.float32)) * dis
    h = jnp.maximum(agg, 0.0)
    o_ref[...] = ((jnp.dot(h, w_ref[...], preferred_element_type=jnp.float32)
                   + b_ref[...]) * dis).astype(jnp.bfloat16)


_layer2_call = pl.pallas_call(
    _layer2_body,
    grid=(GRID,),
    in_specs=[
        pl.BlockSpec((NC, RB, DF), lambda i: (0, i, 0)),
        pl.BlockSpec((RB, DF), lambda i: (i, 0)),
        pl.BlockSpec((RB, 1), lambda i: (i, 0)),
        pl.BlockSpec((DF, DCP), lambda i: (0, 0)),
        pl.BlockSpec((1, DCP), lambda i: (0, 0)),
    ],
    out_specs=pl.BlockSpec((RB, DCP), lambda i: (i, 0)),
    out_shape=jax.ShapeDtypeStruct((NP, DCP), jnp.bfloat16),
)


def _out_body(s_ref, g_ref, dis_ref, o_ref):
    s2 = s_ref[0].astype(jnp.float32) + s_ref[1].astype(jnp.float32)
    z = (s2 + g_ref[...].astype(jnp.float32)) * dis_ref[...]
    col = lax.broadcasted_iota(jnp.int32, (RB, DCP), 1)
    mask = col < DC
    zm = jnp.where(mask, z, -jnp.inf)
    m = jnp.max(zm, axis=1, keepdims=True)
    e = jnp.where(mask, jnp.exp(zm - m), 0.0)
    lse = jnp.log(jnp.sum(e, axis=1, keepdims=True)) + m
    o_ref[...] = (z - lse)[:, :DC]


_out_call = pl.pallas_call(
    _out_body,
    grid=(GRID,),
    in_specs=[
        pl.BlockSpec((NC, RB, DCP), lambda i: (0, i, 0)),
        pl.BlockSpec((RB, DCP), lambda i: (i, 0)),
        pl.BlockSpec((RB, 1), lambda i: (i, 0)),
    ],
    out_specs=pl.BlockSpec((RB, DC), lambda i: (i, 0)),
    out_shape=jax.ShapeDtypeStruct((N, DC), jnp.float32),
)


def kernel(x, edge_index, W1, b1, W2, b2):
    src = edge_index[0].astype(jnp.int32)
    dst = edge_index[1].astype(jnp.int32)
    pad = EP - E
    src_w = jnp.concatenate(
        [src, jnp.zeros((pad,), jnp.int32)]).reshape(NW, CPW, CHUNK)
    dst_w = jnp.concatenate(
        [dst, jnp.full((pad,), N, jnp.int32)]).reshape(NW, CPW, CHUNK)

    x_p = jnp.pad(x, ((0, NP - N), (0, 0)))
    b1r = b1.reshape(1, DF)
    W2p = jnp.pad(W2, ((0, 0), (0, DCP - DC)))
    b2p = jnp.pad(b2, (0, DCP - DC)).reshape(1, DCP)

    deg2 = _deg_kernel(dst_w)
    dega = deg2[0].reshape(NP, 1)
    degb = deg2[1].reshape(NP, 1)

    g1, dis = _layer1_call(x_p, W1, b1r, dega, degb)
    s1 = _scatter_f(g1, src_w, dst_w)
    g2 = _layer2_call(s1, g1, dis, W2p, b2p)
    s2 = _scatter_c(g2, src_w, dst_w)
    return _out_call(s2, g2, dis)
